# baseline (speedup 1.0000x reference)
import jax
import jax.numpy as jnp
from jax import lax
from jax.experimental import pallas as pl
from jax.experimental.pallas import tpu as pltpu
from jax.experimental.pallas import tpu_sc as plsc

BATCH = 16384
EMBED_DIM = 64
NUM_WORKERS = 32
B_PER_W = BATCH // NUM_WORKERS
CHUNK = 64
NCHUNK = B_PER_W // CHUNK
LANES = 16
NSLOT = 4


def _mf_body(uid_hbm, iid_hbm, table_hbm, out_hbm,
             uraw_v, iraw_v, uidx_v, iidx_v, rows_v, part_v, out_v,
             isem, gsem):
    cid = lax.axis_index("c")
    sid = lax.axis_index("s")
    wid = sid * 2 + cid
    base = wid * B_PER_W

    raw_copies = (
        pltpu.async_copy(uid_hbm.at[pl.ds(base, B_PER_W)], uraw_v, isem),
        pltpu.async_copy(iid_hbm.at[pl.ds(base, B_PER_W)], iraw_v, isem),
    )

    def prep(j):
        for m in range(CHUNK // LANES):
            o = j * CHUNK + m * LANES
            uidx_v.at[j][pl.ds(m * LANES, LANES)] = (
                uraw_v[pl.ds(o, LANES)] >> 1)
            iidx_v.at[j][pl.ds(m * LANES, LANES)] = (
                iraw_v[pl.ds(o, LANES)] >> 1)

    def fire(j):
        slot = j % NSLOT
        u = pltpu.async_copy(table_hbm.at[uidx_v.at[j]],
                             rows_v.at[2 * slot], gsem)
        i = pltpu.async_copy(table_hbm.at[iidx_v.at[j]],
                             rows_v.at[2 * slot + 1], gsem)
        return (u, i)

    for c in raw_copies:
        c.wait()
    pending = []
    for j in range(NSLOT):
        prep(j)
        pending.append(fire(j))

    lane_iota = lax.iota(jnp.int32, LANES)

    for j in range(NCHUNK):
        for c in pending[j]:
            c.wait()
        slot = j % NSLOT
        ubuf = rows_v.at[2 * slot]
        ibuf = rows_v.at[2 * slot + 1]

        def pair_body(h, _):
            for half in range(2):
                g = h * 2 + half
                o = j * CHUNK + g * LANES
                uoffs = (uraw_v[pl.ds(o, LANES)] & 1) << 6
                ioffs = (iraw_v[pl.ds(o, LANES)] & 1) << 6
                for m in range(LANES):
                    b = g * LANES + m
                    uo = uoffs[m]
                    io = ioffs[m]
                    acc = None
                    for k in range(0, EMBED_DIM, LANES):
                        u = ubuf[b, pl.ds(uo + k, LANES)]
                        iv = ibuf[b, pl.ds(io + k, LANES)]
                        p = u * iv
                        acc = p if acc is None else acc + p
                    part_v[half * LANES + m] = acc
            for half in range(2):
                g = h * 2 + half
                out_acc = None
                for k in range(LANES):
                    col = plsc.load_gather(
                        part_v, [half * LANES + lane_iota,
                                 jnp.full((LANES,), k, jnp.int32)])
                    out_acc = col if out_acc is None else out_acc + col
                out_v[pl.ds(j * CHUNK + g * LANES, LANES)] = out_acc
            return 0

        lax.fori_loop(0, CHUNK // (2 * LANES), pair_body, 0)

        if j + NSLOT < NCHUNK:
            prep(j + NSLOT)
            pending.append(fire(j + NSLOT))

    pltpu.sync_copy(out_v, out_hbm.at[pl.ds(base, B_PER_W)])


def _mf(uid, iid, table2):
    mesh = plsc.VectorSubcoreMesh(core_axis_name="c", subcore_axis_name="s")
    kern = pl.kernel(
        _mf_body,
        out_type=jax.ShapeDtypeStruct((BATCH,), jnp.float32),
        mesh=mesh,
        scratch_types=[
            pltpu.VMEM((B_PER_W,), jnp.int32),
            pltpu.VMEM((B_PER_W,), jnp.int32),
            pltpu.VMEM((NCHUNK, CHUNK), jnp.int32),
            pltpu.VMEM((NCHUNK, CHUNK), jnp.int32),
            pltpu.VMEM((2 * NSLOT, CHUNK, 128), jnp.float32),
            pltpu.VMEM((2 * LANES, LANES), jnp.float32),
            pltpu.VMEM((B_PER_W,), jnp.float32),
            pltpu.SemaphoreType.DMA,
            pltpu.SemaphoreType.DMA,
        ],
        compiler_params=pltpu.CompilerParams(needs_layout_passes=False,
                                             use_tc_tiling_on_sc=True),
    )
    return kern(uid, iid, table2)


@jax.jit
def _run(userID, ItemID, user_table):
    table2 = jnp.reshape(user_table, (user_table.shape[0] // 2, 128))
    return _mf(userID.astype(jnp.int32), ItemID.astype(jnp.int32), table2)


def kernel(userID, ItemID, user_table, item_table):
    del item_table
    return _run(userID, ItemID, user_table)
